# baseline (speedup 1.0000x reference)
import functools

import jax
import jax.numpy as jnp
from jax import lax
from jax.experimental import pallas as pl
from jax.experimental.pallas import tpu as pltpu
from jax.experimental.pallas import tpu_sc as plsc

_E = 16
_ROWS = 8192
_COLS = 2048
_BLK = 512
_GRID = _ROWS // _BLK
_INV_N = 1.0 / float(_ROWS * _COLS)
_ALPHA = 1.0 / 1000.0
_Z = 0.001


def _sum_body(x_ref, out_ref, acc_ref):
    i = pl.program_id(0)

    @pl.when(i == 0)
    def _init():
        acc_ref[0] = 0.0

    acc_ref[0] += jnp.sum(x_ref[...])

    @pl.when(i == _GRID - 1)
    def _fin():
        ids = lax.broadcasted_iota(jnp.int32, (1, _E), 1)
        out_ref[...] = jnp.where(ids == 0, acc_ref[0], 0.0)


def _tc_partial_sum(wm_state):
    out = pl.pallas_call(
        _sum_body,
        grid=(_GRID,),
        in_specs=[pl.BlockSpec((_BLK, _COLS), lambda i: (i, 0))],
        out_specs=pl.BlockSpec((1, _E), lambda i: (0, 0)),
        out_shape=jax.ShapeDtypeStruct((1, _E), jnp.float32),
        scratch_shapes=[pltpu.SMEM((1,), jnp.float32)],
    )(wm_state)
    return out.reshape(_E)


def _make_router(num_partial_rows: int):
    p = num_partial_rows
    mesh = plsc.VectorSubcoreMesh(core_axis_name="c", subcore_axis_name="s")

    @functools.partial(
        pl.kernel,
        out_type=jax.ShapeDtypeStruct((4 * _E,), jnp.float32),
        mesh=mesh,
        compiler_params=pltpu.CompilerParams(needs_layout_passes=False),
        scratch_types=[
            pltpu.VMEM((p * _E,), jnp.float32),
            pltpu.VMEM((_E,), jnp.float32),
            pltpu.VMEM((_E,), jnp.float32),
            pltpu.VMEM((4 * _E,), jnp.float32),
            pltpu.VMEM((_E,), jnp.float32),
            pltpu.VMEM((_E,), jnp.int32),
        ],
    )
    def router(parts_hbm, proto_hbm, ema_hbm, out_hbm,
               v_parts, v_proto, v_ema, v_out, v_shf, v_shi):
        cid = lax.axis_index("c")
        sid = lax.axis_index("s")

        @pl.when((cid == 0) & (sid == 0))
        def _only_tile0():
            pltpu.sync_copy(parts_hbm, v_parts)
            pltpu.sync_copy(proto_hbm, v_proto)
            pltpu.sync_copy(ema_hbm, v_ema)

            ids = lax.iota(jnp.int32, _E)

            def ar_f(vec, op):
                for sh in (1, 2, 4, 8):
                    v_shf[...] = vec
                    vec = op(vec, plsc.load_gather(v_shf, [ids ^ sh]))
                return vec

            def ar_i(vec, op):
                for sh in (1, 2, 4, 8):
                    v_shi[...] = vec
                    vec = op(vec, plsc.load_gather(v_shi, [ids ^ sh]))
                return vec

            def _acc(c, a):
                return a + v_parts[pl.ds(c * _E, _E)]

            acc = lax.fori_loop(0, p, _acc, jnp.zeros((_E,), jnp.float32))
            x = ar_f(acc, jnp.add) * _INV_N

            t = v_proto[...] - x
            sim = -(t * t)
            m = ar_f(sim, jnp.maximum)
            e = jnp.exp(sim - m)
            probs = e / ar_f(e, jnp.add)
            m1 = ar_f(probs, jnp.maximum)
            i1 = ar_i(jnp.where(probs == m1, ids, _E), jnp.minimum)
            hit1 = ids == i1
            probs2 = jnp.where(hit1, -jnp.inf, probs)
            m2 = ar_f(probs2, jnp.maximum)
            i2 = ar_i(jnp.where(probs2 == m2, ids, _E), jnp.minimum)
            mask = (hit1 | (ids == i2)).astype(jnp.float32)
            usage = (1.0 - _ALPHA) * v_ema[...] + _ALPHA * mask
            d = usage - (1.0 / _E)
            loss = ar_f(d * d, jnp.add) * ((1.0 / _E) * _Z)
            misc = jnp.where(
                ids == 0, loss,
                jnp.where(ids == 1, i1.astype(jnp.float32),
                          jnp.where(ids == 2, i2.astype(jnp.float32), 0.0)))
            v_out[pl.ds(0, _E)] = mask
            v_out[pl.ds(_E, _E)] = probs
            v_out[pl.ds(2 * _E, _E)] = usage
            v_out[pl.ds(3 * _E, _E)] = misc
            pltpu.sync_copy(v_out, out_hbm)

    return router


_router_1 = _make_router(1)


@jax.jit
def kernel(wm_state, prototypes, usage_ema):
    parts = _tc_partial_sum(wm_state)
    packed = _router_1(parts, prototypes.reshape(_E), usage_ema)
    mask = packed[0:_E]
    probs = packed[_E:2 * _E]
    usage = packed[2 * _E:3 * _E]
    loss = packed[3 * _E]
    idx = packed[3 * _E + 1:3 * _E + 3].astype(jnp.int32)
    return (mask, probs, loss, idx, usage)


# speedup vs baseline: 1.1782x; 1.1782x over previous
import functools

import jax
import jax.numpy as jnp
from jax import lax
from jax.experimental import pallas as pl
from jax.experimental.pallas import tpu as pltpu
from jax.experimental.pallas import tpu_sc as plsc

_E = 16
_ROWS = 8192
_COLS = 2048
_BLK = 512
_GRID = _ROWS // _BLK
_INV_N = 1.0 / float(_ROWS * _COLS)
_ALPHA = 1.0 / 1000.0
_Z = 0.001


_GRID2 = _ROWS // (2 * _BLK)


def _sum_body(a_ref, b_ref, out_ref, acc_ref):
    i = pl.program_id(0)

    @pl.when(i == 0)
    def _init():
        acc_ref[0] = 0.0

    acc_ref[0] += jnp.sum(a_ref[...]) + jnp.sum(b_ref[...])

    @pl.when(i == _GRID2 - 1)
    def _fin():
        ids = lax.broadcasted_iota(jnp.int32, (1, _E), 1)
        out_ref[...] = jnp.where(ids == 0, acc_ref[0], 0.0)


def _tc_partial_sum(wm_state):
    wm3d = wm_state.reshape(_ROWS // _BLK, _BLK, _COLS)
    out = pl.pallas_call(
        _sum_body,
        grid=(_GRID2,),
        in_specs=[
            pl.BlockSpec((_BLK, _COLS), lambda i: (2 * i, 0)),
            pl.BlockSpec((1, _BLK, _COLS), lambda i: (2 * i + 1, 0, 0)),
        ],
        out_specs=pl.BlockSpec((1, _E), lambda i: (0, 0)),
        out_shape=jax.ShapeDtypeStruct((1, _E), jnp.float32),
        scratch_shapes=[pltpu.SMEM((1,), jnp.float32)],
    )(wm_state, wm3d)
    return out.reshape(_E)


def _make_router(num_partial_rows: int):
    p = num_partial_rows
    mesh = plsc.VectorSubcoreMesh(core_axis_name="c", subcore_axis_name="s")

    @functools.partial(
        pl.kernel,
        out_type=jax.ShapeDtypeStruct((4 * _E,), jnp.float32),
        mesh=mesh,
        compiler_params=pltpu.CompilerParams(needs_layout_passes=False),
        scratch_types=[
            pltpu.VMEM((p * _E,), jnp.float32),
            pltpu.VMEM((_E,), jnp.float32),
            pltpu.VMEM((_E,), jnp.float32),
            pltpu.VMEM((4 * _E,), jnp.float32),
            pltpu.VMEM((_E,), jnp.float32),
            pltpu.VMEM((_E,), jnp.int32),
        ],
    )
    def router(parts_hbm, proto_hbm, ema_hbm, out_hbm,
               v_parts, v_proto, v_ema, v_out, v_shf, v_shi):
        cid = lax.axis_index("c")
        sid = lax.axis_index("s")

        @pl.when((cid == 0) & (sid == 0))
        def _only_tile0():
            pltpu.sync_copy(parts_hbm, v_parts)
            pltpu.sync_copy(proto_hbm, v_proto)
            pltpu.sync_copy(ema_hbm, v_ema)

            ids = lax.iota(jnp.int32, _E)

            def ar_f(vec, op):
                for sh in (1, 2, 4, 8):
                    v_shf[...] = vec
                    vec = op(vec, plsc.load_gather(v_shf, [ids ^ sh]))
                return vec

            def ar_i(vec, op):
                for sh in (1, 2, 4, 8):
                    v_shi[...] = vec
                    vec = op(vec, plsc.load_gather(v_shi, [ids ^ sh]))
                return vec

            def _acc(c, a):
                return a + v_parts[pl.ds(c * _E, _E)]

            acc = lax.fori_loop(0, p, _acc, jnp.zeros((_E,), jnp.float32))
            x = ar_f(acc, jnp.add) * _INV_N

            t = v_proto[...] - x
            sim = -(t * t)
            m = ar_f(sim, jnp.maximum)
            e = jnp.exp(sim - m)
            probs = e / ar_f(e, jnp.add)
            m1 = ar_f(probs, jnp.maximum)
            i1 = ar_i(jnp.where(probs == m1, ids, _E), jnp.minimum)
            hit1 = ids == i1
            probs2 = jnp.where(hit1, -jnp.inf, probs)
            m2 = ar_f(probs2, jnp.maximum)
            i2 = ar_i(jnp.where(probs2 == m2, ids, _E), jnp.minimum)
            mask = (hit1 | (ids == i2)).astype(jnp.float32)
            usage = (1.0 - _ALPHA) * v_ema[...] + _ALPHA * mask
            d = usage - (1.0 / _E)
            loss = ar_f(d * d, jnp.add) * ((1.0 / _E) * _Z)
            misc = jnp.where(
                ids == 0, loss,
                jnp.where(ids == 1, i1.astype(jnp.float32),
                          jnp.where(ids == 2, i2.astype(jnp.float32), 0.0)))
            v_out[pl.ds(0, _E)] = mask
            v_out[pl.ds(_E, _E)] = probs
            v_out[pl.ds(2 * _E, _E)] = usage
            v_out[pl.ds(3 * _E, _E)] = misc
            pltpu.sync_copy(v_out, out_hbm)

    return router


_router_1 = _make_router(1)


@jax.jit
def kernel(wm_state, prototypes, usage_ema):
    parts = _tc_partial_sum(wm_state)
    packed = _router_1(parts, prototypes.reshape(_E), usage_ema)
    mask = packed[0:_E]
    probs = packed[_E:2 * _E]
    usage = packed[2 * _E:3 * _E]
    loss = packed[3 * _E]
    idx = packed[3 * _E + 1:3 * _E + 3].astype(jnp.int32)
    return (mask, probs, loss, idx, usage)
